# baseline (speedup 1.0000x reference)
import jax
import jax.numpy as jnp
from jax import lax
from jax.experimental import pallas as pl
from jax.experimental.pallas import tpu as pltpu
from jax.experimental.pallas import tpu_sc as plsc

_B4, _S, _D = 4096, 26, 64
_B = _B4 * _S
_NC, _NS = 2, 16
_NW = _NC * _NS
_BPW = _B // _NW
_CHUNK = 128
_NCHUNK = _BPW // _CHUNK


def _gather_body(idx_hbm, table_hbm, out_hbm, idx_v, rows_v, gsem):
    wid = lax.axis_index("s") * _NC + lax.axis_index("c")
    base = wid * _BPW
    pltpu.sync_copy(idx_hbm.at[wid], idx_v)

    def step(j, carry):
        pltpu.async_copy(table_hbm.at[idx_v.at[j]], rows_v, gsem).wait()
        pltpu.sync_copy(rows_v, out_hbm.at[pl.ds(base + j * _CHUNK, _CHUNK)])
        return carry

    lax.fori_loop(0, _NCHUNK, step, 0)


@jax.jit
def _gather(idx, table):
    mesh = plsc.VectorSubcoreMesh(core_axis_name="c", subcore_axis_name="s")
    f = pl.kernel(
        _gather_body,
        out_type=jax.ShapeDtypeStruct((_B, _D), jnp.float32),
        mesh=mesh,
        scratch_types=[
            pltpu.VMEM((_NCHUNK, _CHUNK), jnp.int32),
            pltpu.VMEM((_CHUNK, _D), jnp.float32),
            pltpu.SemaphoreType.DMA,
        ],
        compiler_params=pltpu.CompilerParams(use_tc_tiling_on_sc=False),
    )
    return f(idx, table)


def kernel(x, weight):
    idx = x.astype(jnp.int32).reshape(_NW, _NCHUNK, _CHUNK)
    out = _gather(idx, weight)
    return out.reshape(_B4, _S, _D)


# speedup vs baseline: 1.0226x; 1.0226x over previous
import jax
import jax.numpy as jnp
from jax import lax
from jax.experimental import pallas as pl
from jax.experimental.pallas import tpu as pltpu
from jax.experimental.pallas import tpu_sc as plsc

_B4, _S, _D = 4096, 26, 64
_B = _B4 * _S
_NC, _NS = 2, 16
_NW = _NC * _NS
_BPW = _B // _NW
_CHUNK = 104
_NCHUNK = _BPW // _CHUNK
_NBUF = 4
_NR = _NCHUNK // _NBUF


def _gather_body(idx_hbm, table_hbm, out_hbm, idx_v, rows_v,
                 gs0, gs1, gs2, gs3, ss0, ss1, ss2, ss3):
    gsems = (gs0, gs1, gs2, gs3)
    ssems = (ss0, ss1, ss2, ss3)
    wid = lax.axis_index("s") * _NC + lax.axis_index("c")
    base = wid * _BPW
    pltpu.sync_copy(idx_hbm.at[wid], idx_v)

    def g_copy(j, b):
        return pltpu.make_async_copy(
            table_hbm.at[idx_v.at[j]], rows_v.at[b], gsems[b])

    def s_copy(j, b):
        return pltpu.make_async_copy(
            rows_v.at[b], out_hbm.at[pl.ds(base + j * _CHUNK, _CHUNK)],
            ssems[b])

    for b in range(_NBUF):
        g_copy(b, b).start()

    def round_body(r, carry):
        for b in range(_NBUF):
            g_copy((r - 1) * _NBUF + b, b).wait()
            s_copy((r - 1) * _NBUF + b, b).start()
        for b in range(_NBUF):
            s_copy((r - 1) * _NBUF + b, b).wait()
            g_copy(r * _NBUF + b, b).start()
        return carry

    lax.fori_loop(1, _NR, round_body, 0, unroll=False)

    for b in range(_NBUF):
        g_copy((_NR - 1) * _NBUF + b, b).wait()
        s_copy((_NR - 1) * _NBUF + b, b).start()
    for b in range(_NBUF):
        s_copy((_NR - 1) * _NBUF + b, b).wait()


@jax.jit
def _gather(idx, table):
    mesh = plsc.VectorSubcoreMesh(core_axis_name="c", subcore_axis_name="s")
    f = pl.kernel(
        _gather_body,
        out_type=jax.ShapeDtypeStruct((_B, _D), jnp.float32),
        mesh=mesh,
        scratch_types=[
            pltpu.VMEM((_NCHUNK, _CHUNK), jnp.int32),
            pltpu.VMEM((_NBUF, _CHUNK, _D), jnp.float32),
        ] + [pltpu.SemaphoreType.DMA] * (2 * _NBUF),
        compiler_params=pltpu.CompilerParams(use_tc_tiling_on_sc=False),
    )
    return f(idx, table)


def kernel(x, weight):
    idx = x.astype(jnp.int32).reshape(_NW, _NCHUNK, _CHUNK)
    out = _gather(idx, weight)
    return out.reshape(_B4, _S, _D)
